# baseline (speedup 1.0000x reference)
import functools

import jax
import jax.numpy as jnp
from jax import lax
from jax.experimental import pallas as pl
from jax.experimental.pallas import tpu as pltpu
from jax.experimental.pallas import tpu_sc as plsc

BATCH = 4096
SEQ = 200
EMBED = 32
TOTAL = BATCH * SEQ
VOCAB = 1000000

NUM_CORES = 2
NUM_SUBCORES = 16
NW = NUM_CORES * NUM_SUBCORES
CHUNK = 256
RS_PER_CHUNK = CHUNK // 128
NCHUNK_TOTAL = TOTAL // CHUNK
PER_WORKER = NCHUNK_TOTAL // NW


def _emb_body(idx_hbm, table_hbm, out_hbm, *scr):
    idx_v = scr[0:2]
    g = scr[2:4]
    tbuf = scr[4:6]
    gsem = scr[6:8]
    ssem = scr[8:10]

    wid = lax.axis_index("s") * NUM_CORES + lax.axis_index("c")
    c0 = wid * PER_WORKER

    iota = lax.iota(jnp.int32, 16)
    te0 = iota // 8
    te1 = te0 + 2
    re_v = iota % 8

    def gstart(j, b):
        pltpu.sync_copy(idx_hbm.at[pl.ds((c0 + j) * CHUNK, CHUNK)], idx_v[b])
        pltpu.make_async_copy(table_hbm.at[idx_v[b]], g[b], gsem[b]).start()

    def gwait(b):
        pltpu.make_async_copy(table_hbm.at[idx_v[b]], g[b], gsem[b]).wait()

    def sdesc(j, b):
        cj = c0 + j
        p = cj // 4
        h = cj % 4
        ts = p // 32
        tb = p % 32
        s0 = ts * 8 + h * 2
        return pltpu.make_async_copy(
            tbuf[b].at[:, :, :, pl.ds(0, 128)],
            out_hbm.at[pl.ds(s0, RS_PER_CHUNK), :, tb], ssem[b])

    z16 = iota * 0

    def transpose(b):
        for rs_l in range(RS_PER_CHUNK):
            rs_f = jnp.full((16,), rs_l, jnp.int32)

            @pl.loop(0, 128, unroll=8)
            def _(rb):
                r = rs_l * 128 + rb
                v0 = g[b][r, pl.ds(0, 16)]
                v1 = g[b][r, pl.ds(16, 16)]
                rb_f = z16 + rb
                plsc.store_scatter(tbuf[b], [rs_f, te0, re_v, rb_f], v0)
                plsc.store_scatter(tbuf[b], [rs_f, te1, re_v, rb_f], v1)

    gstart(0, 0)

    @pl.loop(0, PER_WORKER, step=2)
    def _(i):
        for b in range(2):
            j = i + b
            ob = 1 - b

            @pl.when(j + 1 <= PER_WORKER - 1)
            def _():
                gstart(j + 1, ob)

            gwait(b)

            @pl.when(j >= 2)
            def _():
                sdesc(0, b).wait()

            transpose(b)
            sdesc(j, b).start()

    for b in range(2):
        sdesc(0, b).wait()


@jax.jit
def _embedding_lookup(x_flat, table):
    mesh = plsc.VectorSubcoreMesh(core_axis_name="c", subcore_axis_name="s")
    kern = functools.partial(
        pl.kernel,
        mesh=mesh,
        out_type=jax.ShapeDtypeStruct((SEQ, 4, BATCH // 128, 8, 128),
                                      jnp.float32),
        scratch_types=(
            [pltpu.VMEM((CHUNK,), jnp.int32)] * 2
            + [pltpu.VMEM((CHUNK, 128), jnp.float32)] * 2
            + [pltpu.VMEM((RS_PER_CHUNK, 4, 8, 129), jnp.float32)] * 2
            + [pltpu.SemaphoreType.DMA] * 4
        ),
        compiler_params=pltpu.CompilerParams(use_tc_tiling_on_sc=False,
                                             needs_layout_passes=False),
    )(_emb_body)
    return kern(x_flat, table)


def kernel(x, W_embed):
    xt = (x.astype(jnp.int32)
          .reshape(BATCH // 128, 128, SEQ // 8, 8)
          .transpose(2, 0, 3, 1)
          .reshape(TOTAL))
    W128 = jnp.broadcast_to(W_embed[:, None, :],
                            (VOCAB, 4, EMBED)).reshape(VOCAB, 128)
    out5 = _embedding_lookup(xt, W128)
    return (out5.transpose(2, 4, 0, 1, 3)
            .reshape(BATCH, SEQ, EMBED))


# speedup vs baseline: 1.2685x; 1.2685x over previous
import functools

import jax
import jax.numpy as jnp
from jax import lax
from jax.experimental import pallas as pl
from jax.experimental.pallas import tpu as pltpu
from jax.experimental.pallas import tpu_sc as plsc

BATCH = 4096
SEQ = 200
EMBED = 32
TOTAL = BATCH * SEQ

NUM_CORES = 2
NUM_SUBCORES = 16
NW = NUM_CORES * NUM_SUBCORES
CHUNK = 512
RS_PER_CHUNK = CHUNK // 128
NCHUNK_TOTAL = TOTAL // CHUNK
PER_WORKER = NCHUNK_TOTAL // NW


def _emb_body(idx_hbm, table_hbm, out_hbm, *scr):
    idx_all = scr[0]
    g = scr[1:3]
    tbuf = scr[3:5]
    gsem = scr[5:7]
    ssem = scr[7:9]

    wid = lax.axis_index("s") * NUM_CORES + lax.axis_index("c")
    c0 = wid * PER_WORKER

    iota = lax.iota(jnp.int32, 16)
    te0 = iota // 8
    te1 = te0 + 2
    re_v = iota % 8

    pltpu.sync_copy(idx_hbm.at[pl.ds(c0 * CHUNK, PER_WORKER * CHUNK)], idx_all)

    def gstart(j, b):
        pltpu.make_async_copy(
            table_hbm.at[idx_all.at[pl.ds(j * CHUNK, CHUNK)]],
            g[b], gsem[b]).start()

    def gwait(b):
        pltpu.make_async_copy(
            table_hbm.at[idx_all.at[pl.ds(0, CHUNK)]], g[b], gsem[b]).wait()

    def sdesc(j, b):
        cj = c0 + j
        p = cj // 2
        h = cj % 2
        ts = p // 32
        tb = p % 32
        s0 = ts * 8 + h * 4
        return pltpu.make_async_copy(
            tbuf[b].at[:, :, :, pl.ds(0, 128)],
            out_hbm.at[pl.ds(s0, RS_PER_CHUNK), :, tb], ssem[b])

    z16 = iota * 0

    def transpose(b):
        for rs_l in range(RS_PER_CHUNK):
            rs_f = jnp.full((16,), rs_l, jnp.int32)

            @pl.loop(0, 128, unroll=8)
            def _(rb):
                r = rs_l * 128 + rb
                v0 = g[b][r, pl.ds(0, 16)]
                v1 = g[b][r, pl.ds(16, 16)]
                rb_f = z16 + rb
                plsc.store_scatter(tbuf[b], [rs_f, te0, re_v, rb_f], v0)
                plsc.store_scatter(tbuf[b], [rs_f, te1, re_v, rb_f], v1)

    gstart(0, 0)

    @pl.loop(0, PER_WORKER, step=2)
    def _(i):
        for b in range(2):
            j = i + b
            ob = 1 - b

            @pl.when(j + 1 <= PER_WORKER - 1)
            def _():
                gstart(j + 1, ob)

            gwait(b)

            @pl.when(j >= 2)
            def _():
                sdesc(0, b).wait()

            transpose(b)
            sdesc(j, b).start()

    for b in range(2):
        sdesc(0, b).wait()


@jax.jit
def _embedding_lookup(x_flat, table):
    mesh = plsc.VectorSubcoreMesh(core_axis_name="c", subcore_axis_name="s")
    kern = functools.partial(
        pl.kernel,
        mesh=mesh,
        out_type=jax.ShapeDtypeStruct((SEQ, 4, BATCH // 128, 8, 128),
                                      jnp.float32),
        scratch_types=(
            [pltpu.VMEM((PER_WORKER * CHUNK,), jnp.int32)]
            + [pltpu.VMEM((CHUNK, EMBED), jnp.float32)] * 2
            + [pltpu.VMEM((RS_PER_CHUNK, 4, 8, 129), jnp.float32)] * 2
            + [pltpu.SemaphoreType.DMA] * 4
        ),
        compiler_params=pltpu.CompilerParams(use_tc_tiling_on_sc=False,
                                             needs_layout_passes=False),
    )(_emb_body)
    return kern(x_flat, table)


def kernel(x, W_embed):
    xt = (x.astype(jnp.int32)
          .reshape(BATCH // 128, 128, SEQ // 8, 8)
          .transpose(2, 0, 3, 1)
          .reshape(TOTAL))
    out5 = _embedding_lookup(xt, W_embed)
    return (out5.transpose(2, 4, 0, 1, 3)
            .reshape(BATCH, SEQ, EMBED))
